# baseline (speedup 1.0000x reference)
import functools

import jax
import jax.numpy as jnp
from jax import lax
from jax.experimental import pallas as pl
from jax.experimental.pallas import tpu as pltpu
from jax.experimental.pallas import tpu_sc as plsc

_ACTION_DIM = 1024
_SET_COLS = (7, 42, 123)
_BLOCK_ROWS = 1024

_NUM_CORES = 2
_NUM_SUBCORES = 16
_NUM_WORKERS = _NUM_CORES * _NUM_SUBCORES
_CHUNK_ROWS = 64


def _fill_body(probs_ref):
    col = jax.lax.broadcasted_iota(jnp.int32, probs_ref.shape, 1)
    hit = (col == _SET_COLS[0]) | (col == _SET_COLS[1]) | (col == _SET_COLS[2])
    probs_ref[...] = hit.astype(jnp.float32)


@functools.cache
def _make_sc_copy(rows, feat, dtype):
    rows_per_w = rows // _NUM_WORKERS
    n_chunks = rows_per_w // _CHUNK_ROWS

    @functools.partial(
        pl.kernel,
        out_type=jax.ShapeDtypeStruct((rows, feat), dtype),
        mesh=plsc.VectorSubcoreMesh(
            core_axis_name="c", subcore_axis_name="s"),
        scratch_types=[
            pltpu.VMEM((_CHUNK_ROWS, feat), dtype),
            pltpu.VMEM((_CHUNK_ROWS, feat), dtype),
            pltpu.SemaphoreType.DMA,
            pltpu.SemaphoreType.DMA,
            pltpu.SemaphoreType.DMA,
            pltpu.SemaphoreType.DMA,
        ],
    )
    def sc_copy(hid_hbm, out_hbm, buf0, buf1, rs0, rs1, ws0, ws1):
        wid = lax.axis_index("s") * _NUM_CORES + lax.axis_index("c")
        base = wid * rows_per_w
        bufs = (buf0, buf1)
        rsems = (rs0, rs1)
        wsems = (ws0, ws1)
        reads = [None, None]
        writes = [None, None]
        reads[0] = pltpu.async_copy(
            hid_hbm.at[pl.ds(base, _CHUNK_ROWS)], buf0, rs0)
        for j in range(n_chunks):
            b = j % 2
            nb = (j + 1) % 2
            if j + 1 < n_chunks:
                if writes[nb] is not None:
                    writes[nb].wait()
                reads[nb] = pltpu.async_copy(
                    hid_hbm.at[pl.ds(base + (j + 1) * _CHUNK_ROWS, _CHUNK_ROWS)],
                    bufs[nb], rsems[nb])
            reads[b].wait()
            writes[b] = pltpu.async_copy(
                bufs[b], out_hbm.at[pl.ds(base + j * _CHUNK_ROWS, _CHUNK_ROWS)],
                wsems[b])
        writes[0].wait()
        writes[1].wait()

    return sc_copy


def kernel(hidden, obs, done):
    rows = obs.shape[1]
    probs = pl.pallas_call(
        _fill_body,
        grid=(rows // _BLOCK_ROWS,),
        out_specs=pl.BlockSpec((_BLOCK_ROWS, _ACTION_DIM), lambda i: (i, 0)),
        out_shape=jax.ShapeDtypeStruct((rows, _ACTION_DIM), jnp.float32),
    )()
    hidden_out = _make_sc_copy(hidden.shape[0], hidden.shape[1],
                               hidden.dtype)(hidden)
    return (hidden_out, probs, jnp.asarray(0))
